# baseline (speedup 1.0000x reference)
import jax
import jax.numpy as jnp
from jax import lax
from jax.experimental import pallas as pl
from jax.experimental.pallas import tpu as pltpu
from jax.experimental.pallas import tpu_sc as plsc

N = 10000
E = 320000
D = 128

NC = 2
NS = 16
NW = NC * NS
CH = 128
NCH = 80
EPT = NCH * CH
E_PAD = NW * EPT
N_PAD = 10240
ROWS_PT = N_PAD // NS

_MESH = plsc.VectorSubcoreMesh(core_axis_name="c", subcore_axis_name="s")


def _sc_deg_body(pkw_hbm, zeros1_hbm, deg_hbm, pk_v, dst_all, ones_v,
                 deg_sp, sem):
    cid = lax.axis_index("c")
    sid = lax.axis_index("s")
    wid = cid * NS + sid
    for i in range(CH // 16):
        ones_v[pl.ds(i * 16, 16)] = jnp.full((16,), 1.0, jnp.float32)
    pltpu.sync_copy(zeros1_hbm.at[pl.ds(sid * ROWS_PT, ROWS_PT)],
                    deg_sp.at[pl.ds(sid * ROWS_PT, ROWS_PT)])
    pltpu.sync_copy(pkw_hbm.at[wid], pk_v)

    def unpack(r, carry):
        for j in range(CH // 16):
            v = pk_v[r, pl.ds(j * 16, 16)]
            dst_all[r, pl.ds(j * 16, 16)] = jnp.bitwise_and(v, 0xFFFF)
        return carry

    lax.fori_loop(0, NCH, unpack, 0)
    plsc.subcore_barrier()

    def fire(k, carry):
        pltpu.async_copy(ones_v, deg_sp.at[dst_all.at[k]], sem, add=True)
        return carry

    lax.fori_loop(0, NCH, fire, 0)

    def drain(k, carry):
        pltpu.make_async_copy(ones_v, deg_sp.at[dst_all.at[k]], sem).wait()
        return carry

    lax.fori_loop(0, NCH, drain, 0)
    plsc.subcore_barrier()
    pltpu.sync_copy(deg_sp.at[pl.ds(sid * ROWS_PT, ROWS_PT)],
                    deg_hbm.at[cid, pl.ds(sid * ROWS_PT, ROWS_PT)])


_sc_deg = pl.kernel(
    _sc_deg_body,
    out_type=jax.ShapeDtypeStruct((NC, N_PAD), jnp.float32),
    mesh=_MESH,
    scratch_types=[
        pltpu.VMEM((NCH, CH), jnp.int32),
        pltpu.VMEM((NCH, CH), jnp.int32),
        pltpu.VMEM((CH,), jnp.float32),
        pltpu.VMEM_SHARED((N_PAD,), jnp.float32),
        pltpu.SemaphoreType.DMA,
    ],
)


def _sc_agg_body(pkw_hbm, xs_hbm, zeros2_hbm, z_hbm,
                 pk_v, srcb, dstb, rows, z_sp, s0, s1):
    sems = [s0, s1]
    cid = lax.axis_index("c")
    sid = lax.axis_index("s")
    wid = cid * NS + sid
    pltpu.sync_copy(zeros2_hbm.at[pl.ds(sid * ROWS_PT, ROWS_PT)],
                    z_sp.at[pl.ds(sid * ROWS_PT, ROWS_PT)])
    pltpu.sync_copy(pkw_hbm.at[wid], pk_v)
    plsc.subcore_barrier()

    def unpack(c, b):
        for j in range(CH // 16):
            v = pk_v[c, pl.ds(j * 16, 16)]
            srcb[b, pl.ds(j * 16, 16)] = jnp.right_shift(v, 16)
            dstb[b, pl.ds(j * 16, 16)] = jnp.bitwise_and(v, 0xFFFF)

    def start_gather(b):
        pltpu.async_copy(xs_hbm.at[srcb.at[b]], rows.at[b], sems[b])

    def wait_gather(b):
        pltpu.make_async_copy(xs_hbm.at[srcb.at[b]], rows.at[b],
                              sems[b]).wait()

    def scatter(b):
        pltpu.sync_copy(rows.at[b], z_sp.at[dstb.at[b]], add=True)

    def step(c, b, nxt):
        wait_gather(b)
        if nxt is not None:
            unpack(nxt, 1 - b)
            start_gather(1 - b)
        scatter(b)

    unpack(0, 0)
    start_gather(0)

    def group(g, carry):
        step(2 * g, 0, 2 * g + 1)
        step(2 * g + 1, 1, 2 * g + 2)
        return carry

    lax.fori_loop(0, NCH // 2 - 1, group, 0)
    step(NCH - 2, 0, NCH - 1)
    step(NCH - 1, 1, None)
    plsc.subcore_barrier()
    pltpu.sync_copy(z_sp.at[pl.ds(sid * ROWS_PT, ROWS_PT)],
                    z_hbm.at[cid, pl.ds(sid * ROWS_PT, ROWS_PT)])


_sc_agg = pl.kernel(
    _sc_agg_body,
    out_type=jax.ShapeDtypeStruct((NC, N_PAD, D), jnp.float32),
    mesh=_MESH,
    scratch_types=[
        pltpu.VMEM((NCH, CH), jnp.int32),
        pltpu.VMEM((2, CH), jnp.int32),
        pltpu.VMEM((2, CH), jnp.int32),
        pltpu.VMEM((2, CH, D), jnp.float32),
        pltpu.VMEM_SHARED((N_PAD, D), jnp.float32),
        pltpu.SemaphoreType.DMA,
        pltpu.SemaphoreType.DMA,
    ],
)


def _tc_xs_body(x_ref, degp_ref, xs_ref):
    deg = degp_ref[0, :N] + degp_ref[1, :N] + 1.0
    dinv = lax.rsqrt(deg)
    xs_ref[...] = x_ref[...] * dinv[:, None]


_tc_xs = pl.pallas_call(
    _tc_xs_body,
    out_shape=jax.ShapeDtypeStruct((N, D), jnp.float32),
)


def _tc_out_body(x_ref, zp_ref, degp_ref, w1_ref, b1_ref, w2_ref, b2_ref,
                 o_ref):
    deg = degp_ref[0, :N] + degp_ref[1, :N] + 1.0
    dinv = lax.rsqrt(deg)[:, None]
    z = (zp_ref[0, :N] + zp_ref[1, :N]) * dinv + x_ref[...] * (dinv * dinv)
    h1 = jnp.dot(z, w1_ref[...], preferred_element_type=jnp.float32)
    h1 = jnp.maximum(h1 + b1_ref[...], 0.0)
    h2 = jnp.dot(z, w2_ref[...], preferred_element_type=jnp.float32)
    o_ref[...] = h1 + h2 + b2_ref[...]


_tc_out = pl.pallas_call(
    _tc_out_body,
    out_shape=jax.ShapeDtypeStruct((N, D), jnp.float32),
)


def kernel(x, edge_index, W1, b1, W2, b2):
    ei = edge_index.astype(jnp.int32)
    pad = E_PAD - E
    src = jnp.concatenate([ei[0], jnp.zeros((pad,), jnp.int32)])
    dst = jnp.concatenate([ei[1], jnp.full((pad,), N, jnp.int32)])
    pkw = jnp.bitwise_or(jnp.left_shift(src, 16), dst).reshape(NW, NCH, CH)
    zeros1 = jnp.zeros((N_PAD,), jnp.float32)
    zeros2 = jnp.zeros((N_PAD, D), jnp.float32)
    degp = _sc_deg(pkw, zeros1)
    xs = _tc_xs(x, degp)
    zp = _sc_agg(pkw, xs, zeros2)
    return _tc_out(x, zp, degp, W1, b1, W2, b2)


# speedup vs baseline: 6.9828x; 6.9828x over previous
import jax
import jax.numpy as jnp
from jax import lax
from jax.experimental import pallas as pl
from jax.experimental.pallas import tpu as pltpu
from jax.experimental.pallas import tpu_sc as plsc

N = 10000
E = 320000
D = 128

NC = 2
NS = 16
NW = NC * NS
CH = 64
NCH = 160
NP = 2
NCHP = NCH // NP
NR = 4
EPT = NCH * CH
E_PAD = NW * EPT
N_PAD = 10240
ROWS_PT = N_PAD // NS

_MESH = plsc.VectorSubcoreMesh(core_axis_name="c", subcore_axis_name="s")


def _sc_deg_body(pkw_hbm, zeros1_hbm, deg_hbm, pk_v, dst_all, ones_v,
                 deg_sp, sem):
    cid = lax.axis_index("c")
    sid = lax.axis_index("s")
    wid = cid * NS + sid
    for i in range(CH // 16):
        ones_v[pl.ds(i * 16, 16)] = jnp.full((16,), 1.0, jnp.float32)
    pltpu.sync_copy(zeros1_hbm.at[pl.ds(sid * ROWS_PT, ROWS_PT)],
                    deg_sp.at[pl.ds(sid * ROWS_PT, ROWS_PT)])
    pltpu.sync_copy(pkw_hbm.at[wid], pk_v)

    def unpack(r, carry):
        for j in range(CH // 16):
            v = pk_v[r, pl.ds(j * 16, 16)]
            dst_all[r, pl.ds(j * 16, 16)] = jnp.bitwise_and(v, 0xFFFF)
        return carry

    lax.fori_loop(0, NCH, unpack, 0)
    plsc.subcore_barrier()

    def fire(k, carry):
        pltpu.async_copy(ones_v, deg_sp.at[dst_all.at[k]], sem, add=True)
        return carry

    lax.fori_loop(0, NCH, fire, 0)

    def drain(k, carry):
        pltpu.make_async_copy(ones_v, deg_sp.at[dst_all.at[k]], sem).wait()
        return carry

    lax.fori_loop(0, NCH, drain, 0)
    plsc.subcore_barrier()
    pltpu.sync_copy(deg_sp.at[pl.ds(sid * ROWS_PT, ROWS_PT)],
                    deg_hbm.at[cid, pl.ds(sid * ROWS_PT, ROWS_PT)])


_sc_deg = pl.kernel(
    _sc_deg_body,
    out_type=jax.ShapeDtypeStruct((NC, N_PAD), jnp.float32),
    mesh=_MESH,
    scratch_types=[
        pltpu.VMEM((NCH, CH), jnp.int32),
        pltpu.VMEM((NCH, CH), jnp.int32),
        pltpu.VMEM((CH,), jnp.float32),
        pltpu.VMEM_SHARED((N_PAD,), jnp.float32),
        pltpu.SemaphoreType.DMA,
    ],
)


def _sc_agg_body(pkw_hbm, xs_hbm, zeros2_hbm, z_hbm,
                 pk_v, srcb, dstb, rows, z_sp, s0, s1, s2, s3):
    sems = [s0, s1, s2, s3]
    cid = lax.axis_index("c")
    sid = lax.axis_index("s")
    wid = cid * NS + sid
    pltpu.sync_copy(zeros2_hbm.at[pl.ds(sid * ROWS_PT, ROWS_PT)],
                    z_sp.at[pl.ds(sid * ROWS_PT, ROWS_PT)])
    plsc.subcore_barrier()

    def unpack(c, b):
        for j in range(CH // 16):
            v = pk_v[c, pl.ds(j * 16, 16)]
            srcb[b, pl.ds(j * 16, 16)] = jnp.right_shift(v, 16)
            dstb[b, pl.ds(j * 16, 16)] = jnp.bitwise_and(v, 0xFFFF)

    def start_gather(b):
        pltpu.async_copy(xs_hbm.at[srcb.at[b]], rows.at[b], sems[b])

    def wait_gather(b):
        pltpu.make_async_copy(xs_hbm.at[srcb.at[b]], rows.at[b],
                              sems[b]).wait()

    def scatter(b):
        pltpu.sync_copy(rows.at[b], z_sp.at[dstb.at[b]], add=True)

    def step(c, b, nxt):
        wait_gather(b)
        if nxt is not None:
            pb = (b + NR - 1) % NR
            unpack(nxt, pb)
            start_gather(pb)
        scatter(b)

    for p in range(NP):
        pltpu.sync_copy(pkw_hbm.at[wid, pl.ds(p * NCHP, NCHP)], pk_v)
        for c0 in range(NR - 1):
            unpack(c0, c0)
            start_gather(c0)

        def group(g, carry):
            for b in range(NR):
                c = NR * g + b
                step(c, b, c + NR - 1)
            return carry

        lax.fori_loop(0, NCHP // NR - 1, group, 0)
        for b in range(NR):
            c = NCHP - NR + b
            step(c, b, c + NR - 1 if c + NR - 1 < NCHP else None)
    plsc.subcore_barrier()
    pltpu.sync_copy(z_sp.at[pl.ds(sid * ROWS_PT, ROWS_PT)],
                    z_hbm.at[cid, pl.ds(sid * ROWS_PT, ROWS_PT)])


_sc_agg = pl.kernel(
    _sc_agg_body,
    out_type=jax.ShapeDtypeStruct((NC, N_PAD, D), jnp.float32),
    mesh=_MESH,
    scratch_types=[
        pltpu.VMEM((NCHP, CH), jnp.int32),
        pltpu.VMEM((NR, CH), jnp.int32),
        pltpu.VMEM((NR, CH), jnp.int32),
        pltpu.VMEM((NR, CH, D), jnp.float32),
        pltpu.VMEM_SHARED((N_PAD, D), jnp.float32),
        pltpu.SemaphoreType.DMA,
        pltpu.SemaphoreType.DMA,
        pltpu.SemaphoreType.DMA,
        pltpu.SemaphoreType.DMA,
    ],
)


def _tc_xs_body(x_ref, degp_ref, xs_ref):
    deg = degp_ref[0, :N] + degp_ref[1, :N] + 1.0
    dinv = lax.rsqrt(deg)
    xs_ref[...] = x_ref[...] * dinv[:, None]


_tc_xs = pl.pallas_call(
    _tc_xs_body,
    out_shape=jax.ShapeDtypeStruct((N, D), jnp.float32),
)


def _tc_out_body(x_ref, zp_ref, degp_ref, w1_ref, b1_ref, w2_ref, b2_ref,
                 o_ref):
    deg = degp_ref[0, :N] + degp_ref[1, :N] + 1.0
    dinv = lax.rsqrt(deg)[:, None]
    z = (zp_ref[0, :N] + zp_ref[1, :N]) * dinv + x_ref[...] * (dinv * dinv)
    h1 = jnp.dot(z, w1_ref[...], preferred_element_type=jnp.float32)
    h1 = jnp.maximum(h1 + b1_ref[...], 0.0)
    h2 = jnp.dot(z, w2_ref[...], preferred_element_type=jnp.float32)
    o_ref[...] = h1 + h2 + b2_ref[...]


_tc_out = pl.pallas_call(
    _tc_out_body,
    out_shape=jax.ShapeDtypeStruct((N, D), jnp.float32),
)


def kernel(x, edge_index, W1, b1, W2, b2):
    ei = edge_index.astype(jnp.int32)
    pad = E_PAD - E
    src = jnp.concatenate([ei[0], jnp.zeros((pad,), jnp.int32)])
    dst = jnp.concatenate([ei[1], jnp.full((pad,), N, jnp.int32)])
    pkw = jnp.bitwise_or(jnp.left_shift(src, 16), dst).reshape(NW, NCH, CH)
    zeros1 = jnp.zeros((N_PAD,), jnp.float32)
    zeros2 = jnp.zeros((N_PAD, D), jnp.float32)
    degp = _sc_deg(pkw, zeros1)
    xs = _tc_xs(x, degp)
    zp = _sc_agg(pkw, xs, zeros2)
    return _tc_out(x, zp, degp, W1, b1, W2, b2)
